# baseline (speedup 1.0000x reference)
import functools

import jax
import jax.numpy as jnp
from jax import lax
from jax.experimental import pallas as pl
from jax.experimental.pallas import tpu as pltpu
from jax.experimental.pallas import tpu_sc as plsc

NC = 2
NS = 16
NW = NC * NS
CHUNK = 128
MAC = 2 * CHUNK
NBUF = 3


@jax.jit
def _run(x_flat, W):
    N = x_flat.shape[0]
    V, D = W.shape
    n_per_w = N // NW
    n_chunks = n_per_w // CHUNK
    n_mac = n_per_w // MAC
    x3 = x_flat.reshape(NW, n_chunks, CHUNK)

    mesh = plsc.VectorSubcoreMesh(core_axis_name="c", subcore_axis_name="s")

    @functools.partial(
        pl.kernel,
        out_type=jax.ShapeDtypeStruct((N, D), jnp.float32),
        mesh=mesh,
        scratch_types=[
            pltpu.VMEM((n_chunks, CHUNK), jnp.int32),
            [pltpu.VMEM((MAC, D), jnp.float32) for _ in range(NBUF)],
            [pltpu.SemaphoreType.DMA for _ in range(NBUF)],
            [pltpu.SemaphoreType.DMA for _ in range(NBUF)],
        ],
    )
    def k(x_hbm, w_hbm, out_hbm, idx_v, bufs, gsems, wsems):
        cid = lax.axis_index("c")
        sid = lax.axis_index("s")
        wid = sid * NC + cid
        base = wid * n_per_w

        pltpu.sync_copy(x_hbm.at[wid], idx_v)

        def gather(t, b):
            pltpu.async_copy(w_hbm.at[idx_v.at[2 * t]],
                             bufs[b].at[pl.ds(0, CHUNK)], gsems[b])
            pltpu.async_copy(w_hbm.at[idx_v.at[2 * t + 1]],
                             bufs[b].at[pl.ds(CHUNK, CHUNK)], gsems[b])

        def wait_gather(b):
            pltpu.make_async_copy(out_hbm.at[pl.ds(base, MAC)], bufs[b],
                                  gsems[b]).wait()

        def write(t, b):
            pltpu.async_copy(bufs[b],
                             out_hbm.at[pl.ds(base + t * MAC, MAC)],
                             wsems[b])

        def wait_write(b):
            pltpu.make_async_copy(bufs[b], out_hbm.at[pl.ds(base, MAC)],
                                  wsems[b]).wait()

        gather(0, 0)
        gather(1, 1)
        wait_gather(0)
        write(0, 0)
        gather(2, 2)
        wait_gather(1)
        write(1, 1)

        def step(i, _):
            for r in range(NBUF):
                t = 2 + i * NBUF + r
                wait_write(r)
                gather(t + 1, r)
                b = (2 + r) % NBUF
                wait_gather(b)
                write(t, b)
            return 0

        lax.fori_loop(0, (n_mac - 4) // NBUF, step, 0)

        t = n_mac - 2
        wait_write((t + 1) % NBUF)
        gather(t + 1, (t + 1) % NBUF)
        wait_gather(t % NBUF)
        write(t, t % NBUF)
        t = n_mac - 1
        wait_gather(t % NBUF)
        write(t, t % NBUF)

        for b in range(NBUF):
            wait_write(b)

    return k(x3, W)


def kernel(x, W):
    x = x.astype(jnp.int32)
    B, S = x.shape
    D = W.shape[1]
    out = _run(x.reshape(B * S), W)
    return out.reshape(B, S, D)


# speedup vs baseline: 1.0014x; 1.0014x over previous
import functools

import jax
import jax.numpy as jnp
from jax import lax
from jax.experimental import pallas as pl
from jax.experimental.pallas import tpu as pltpu
from jax.experimental.pallas import tpu_sc as plsc

NC = 2
NS = 16
NW = NC * NS
CHUNK = 128
NBUF = 6
LOOK = 3


@jax.jit
def _run(x_flat, W):
    N = x_flat.shape[0]
    V, D = W.shape
    n_per_w = N // NW
    n_chunks = n_per_w // CHUNK
    x3 = x_flat.reshape(NW, n_chunks, CHUNK)

    mesh = plsc.VectorSubcoreMesh(core_axis_name="c", subcore_axis_name="s")

    @functools.partial(
        pl.kernel,
        out_type=jax.ShapeDtypeStruct((N, D), jnp.float32),
        mesh=mesh,
        scratch_types=[
            pltpu.VMEM((n_chunks, CHUNK), jnp.int32),
            [pltpu.VMEM((CHUNK, D), jnp.float32) for _ in range(NBUF)],
            [pltpu.SemaphoreType.DMA for _ in range(NBUF)],
            [pltpu.SemaphoreType.DMA for _ in range(NBUF)],
        ],
    )
    def k(x_hbm, w_hbm, out_hbm, idx_v, bufs, gsems, wsems):
        cid = lax.axis_index("c")
        sid = lax.axis_index("s")
        wid = sid * NC + cid
        base = wid * n_per_w

        pltpu.sync_copy(x_hbm.at[wid], idx_v)

        def gather(j, b):
            pltpu.async_copy(w_hbm.at[idx_v.at[j]], bufs[b], gsems[b])

        def wait_gather(b):
            pltpu.make_async_copy(out_hbm.at[pl.ds(base, CHUNK)], bufs[b],
                                  gsems[b]).wait()

        def write(j, b):
            pltpu.async_copy(bufs[b],
                             out_hbm.at[pl.ds(base + j * CHUNK, CHUNK)],
                             wsems[b])

        def wait_write(b):
            pltpu.make_async_copy(bufs[b], out_hbm.at[pl.ds(base, CHUNK)],
                                  wsems[b]).wait()

        def unit(j, bn, b, write_wait, do_gather):
            if write_wait:
                wait_write(bn)
            if do_gather:
                gather(j + LOOK, bn)
            wait_gather(b)
            write(j, b)

        for j in range(LOOK):
            gather(j, j)
        for j in range(NBUF):
            unit(j, (j + LOOK) % NBUF, j % NBUF, j >= LOOK, True)

        n_steady = (n_chunks - LOOK - NBUF) // NBUF * NBUF

        def step(i, _):
            for r in range(NBUF):
                j = NBUF + i * NBUF + r
                unit(j, (r + LOOK) % NBUF, r, True, True)
            return 0

        lax.fori_loop(0, n_steady // NBUF, step, 0)

        for j in range(NBUF + n_steady, n_chunks - LOOK):
            unit(j, (j + LOOK) % NBUF, j % NBUF, True, True)
        for j in range(n_chunks - LOOK, n_chunks):
            unit(j, 0, j % NBUF, False, False)

        for b in range(NBUF):
            wait_write(b)

    return k(x3, W)


def kernel(x, W):
    x = x.astype(jnp.int32)
    B, S = x.shape
    D = W.shape[1]
    out = _run(x.reshape(B * S), W)
    return out.reshape(B, S, D)


# speedup vs baseline: 1.0083x; 1.0070x over previous
import functools

import jax
import jax.numpy as jnp
from jax import lax
from jax.experimental import pallas as pl
from jax.experimental.pallas import tpu as pltpu
from jax.experimental.pallas import tpu_sc as plsc

NC = 2
NS = 16
NW = NC * NS
CHUNK = 128


@jax.jit
def _run(x_flat, W):
    N = x_flat.shape[0]
    V, D = W.shape
    n_per_w = N // NW
    n_chunks = n_per_w // CHUNK
    x3 = x_flat.reshape(NW, n_chunks, CHUNK)

    mesh = plsc.VectorSubcoreMesh(core_axis_name="c", subcore_axis_name="s")

    @functools.partial(
        pl.kernel,
        out_type=jax.ShapeDtypeStruct((N, D), jnp.float32),
        mesh=mesh,
        scratch_types=[
            pltpu.VMEM((n_chunks, CHUNK), jnp.int32),
            pltpu.VMEM((CHUNK, D), jnp.float32),
            pltpu.VMEM((CHUNK, D), jnp.float32),
            pltpu.SemaphoreType.DMA,
            pltpu.SemaphoreType.DMA,
            pltpu.SemaphoreType.DMA,
            pltpu.SemaphoreType.DMA,
        ],
    )
    def k(x_hbm, w_hbm, out_hbm, idx_v, buf0, buf1, g0, g1, w0, w1):
        cid = lax.axis_index("c")
        sid = lax.axis_index("s")
        wid = sid * NC + cid
        base = wid * n_per_w

        pltpu.sync_copy(x_hbm.at[wid], idx_v)

        bufs = (buf0, buf1)
        gsems = (g0, g1)
        wsems = (w0, w1)

        pltpu.async_copy(w_hbm.at[idx_v.at[0]], buf0, g0)
        pltpu.async_copy(w_hbm.at[idx_v.at[1]], buf1, g1)

        def step(i, _):
            for b in range(2):
                j = 2 * i + b
                pltpu.make_async_copy(w_hbm.at[idx_v.at[0]], bufs[b],
                                      gsems[b]).wait()
                pltpu.async_copy(
                    bufs[b], out_hbm.at[pl.ds(base + j * CHUNK, CHUNK)],
                    wsems[b])
                jn = j + 2

                @pl.when(jn < n_chunks)
                def _():
                    pltpu.make_async_copy(
                        bufs[b], out_hbm.at[pl.ds(base, CHUNK)],
                        wsems[b]).wait()
                    pltpu.async_copy(w_hbm.at[idx_v.at[jn]], bufs[b],
                                     gsems[b])
            return 0

        lax.fori_loop(0, n_chunks // 2, step, 0)
        pltpu.make_async_copy(buf0, out_hbm.at[pl.ds(base, CHUNK)], w0).wait()
        pltpu.make_async_copy(buf1, out_hbm.at[pl.ds(base, CHUNK)], w1).wait()

    return k(x3, W)


def kernel(x, W):
    x = x.astype(jnp.int32)
    B, S = x.shape
    D = W.shape[1]
    out = _run(x.reshape(B * S), W)
    return out.reshape(B, S, D)
